# Initial kernel scaffold; baseline (speedup 1.0000x reference)
import jax
import jax.numpy as jnp
from jax.experimental import pallas as pl


def kernel(x, edge_index, fc0_w, fc0_b, bn0_g, bn0_b, conv_w, W_w, W_b, bn1_g, bn1_b, fco_w, fco_b):
    raise NotImplementedError("write your pallas kernel here")


# speedup vs baseline: 12.5322x; 12.5322x over previous
import functools

import jax
import jax.numpy as jnp
from jax import lax
from jax.experimental import pallas as pl
from jax.experimental.pallas import tpu as pltpu
from jax.experimental.pallas import tpu_sc as plsc

_N = 10000
_E = 320000
_D = 128
_NC = 2
_NS = 16
_NW = _NC * _NS
_CH = 128
_CPT = 80
_EPAD = _NW * _CPT * _CH
_RPT = 632
_RPAD = _NS * _RPT

_mesh = plsc.VectorSubcoreMesh(
    core_axis_name="c", subcore_axis_name="s", num_cores=_NC, num_subcores=_NS)


@functools.partial(
    pl.kernel,
    out_type=jax.ShapeDtypeStruct((_NC, _RPAD, 16), jnp.float32),
    mesh=_mesh,
    scratch_types=[
        pltpu.VMEM((_CPT, _CH), jnp.int32),
        pltpu.VMEM((_CH, 16), jnp.float32),
        pltpu.VMEM_SHARED((_RPAD, 16), jnp.float32),
    ],
)
def _deg_sc(dst_hbm, ones_hbm, zeros_hbm, out_hbm, idx_v, ones_v, acc_sh):
    c = lax.axis_index("c")
    s = lax.axis_index("s")
    wid = s * _NC + c
    pltpu.sync_copy(zeros_hbm.at[pl.ds(s * _RPT, _RPT)],
                    acc_sh.at[pl.ds(s * _RPT, _RPT)])
    pltpu.sync_copy(ones_hbm, ones_v)
    pltpu.sync_copy(dst_hbm.at[wid], idx_v)
    plsc.subcore_barrier()

    def chunk(j, carry):
        pltpu.sync_copy(ones_v, acc_sh.at[idx_v.at[j]], add=True)
        return carry

    lax.fori_loop(0, _CPT, chunk, 0)
    plsc.subcore_barrier()
    pltpu.sync_copy(acc_sh.at[pl.ds(s * _RPT, _RPT)],
                    out_hbm.at[c, pl.ds(s * _RPT, _RPT)])


@functools.partial(
    pl.kernel,
    out_type=jax.ShapeDtypeStruct((_NC, _RPAD, _D), jnp.float32),
    mesh=_mesh,
    scratch_types=[
        pltpu.VMEM((_CPT, _CH), jnp.int32),
        pltpu.VMEM((_CPT, _CH), jnp.int32),
        pltpu.VMEM((_CH, _D), jnp.float32),
        pltpu.VMEM_SHARED((_RPAD, _D), jnp.float32),
        pltpu.SemaphoreType.DMA,
    ],
)
def _gather_sc(gs_hbm, src_hbm, dst_hbm, zeros_hbm, out_hbm,
               src_v, dst_v, rows_v, acc_sh, sem):
    c = lax.axis_index("c")
    s = lax.axis_index("s")
    wid = s * _NC + c
    pltpu.sync_copy(zeros_hbm.at[pl.ds(s * _RPT, _RPT)],
                    acc_sh.at[pl.ds(s * _RPT, _RPT)])
    pltpu.sync_copy(src_hbm.at[wid], src_v)
    pltpu.sync_copy(dst_hbm.at[wid], dst_v)
    plsc.subcore_barrier()

    def chunk(j, carry):
        pltpu.async_copy(gs_hbm.at[src_v.at[j]], rows_v, sem).wait()
        pltpu.sync_copy(rows_v, acc_sh.at[dst_v.at[j]], add=True)
        return carry

    lax.fori_loop(0, _CPT, chunk, 0)
    plsc.subcore_barrier()
    pltpu.sync_copy(acc_sh.at[pl.ds(s * _RPT, _RPT)],
                    out_hbm.at[c, pl.ds(s * _RPT, _RPT)])


def _bn_relu(v, g, b):
    m = jnp.mean(v, axis=0, keepdims=True)
    var = jnp.mean((v - m) ** 2, axis=0, keepdims=True)
    return jnp.maximum((v - m) * lax.rsqrt(var + 1e-5) * g[None, :] + b[None, :],
                       0.0)


def _mm_t(a, w):
    return lax.dot_general(a, w, (((1,), (1,)), ((), ())),
                           preferred_element_type=jnp.float32)


def _pre_body(x_ref, w0_ref, b0_ref, g0_ref, bb0_ref, cw_ref, h_ref, gp_ref):
    h = _bn_relu(_mm_t(x_ref[...], w0_ref[...]) + b0_ref[...][None, :],
                 g0_ref[...], bb0_ref[...])
    h_ref[...] = h
    gp_ref[...] = _mm_t(h, cw_ref[...])


def _scale_body(gp_ref, degp_ref, gs_ref):
    deg = degp_ref[0, : _N, 0:1] + degp_ref[1, : _N, 0:1] + 1.0
    gs_ref[...] = gp_ref[...] * lax.rsqrt(deg)


def _post_body(parts_ref, degp_ref, gs_ref, h_ref, ww_ref, wb_ref, g1_ref,
               b1_ref, fw_ref, fb_ref, out_ref):
    deg = degp_ref[0, : _N, 0:1] + degp_ref[1, : _N, 0:1] + 1.0
    prop = (parts_ref[0, : _N, :] + parts_ref[1, : _N, :] + gs_ref[...]) \
        * lax.rsqrt(deg)
    g2 = _bn_relu(_mm_t(prop, ww_ref[...]) + wb_ref[...][None, :],
                  g1_ref[...], b1_ref[...])
    xjk = jnp.maximum(h_ref[...], g2)
    out_ref[...] = _mm_t(xjk, fw_ref[...]) + fb_ref[...][None, :]


_pre_tc = pl.pallas_call(
    _pre_body,
    out_shape=[jax.ShapeDtypeStruct((_N, _D), jnp.float32),
               jax.ShapeDtypeStruct((_N, _D), jnp.float32)],
)

_scale_tc = pl.pallas_call(
    _scale_body,
    out_shape=jax.ShapeDtypeStruct((_N, _D), jnp.float32),
)

_post_tc = pl.pallas_call(
    _post_body,
    out_shape=jax.ShapeDtypeStruct((_N, _D), jnp.float32),
)


def kernel(x, edge_index, fc0_w, fc0_b, bn0_g, bn0_b, conv_w, W_w, W_b,
           bn1_g, bn1_b, fco_w, fco_b):
    src = edge_index[0].astype(jnp.int32)
    dst = edge_index[1].astype(jnp.int32)
    pad = _EPAD - _E
    srcp = jnp.concatenate([src, jnp.zeros((pad,), jnp.int32)])
    dstp = jnp.concatenate([dst, jnp.full((pad,), _N, jnp.int32)])
    srcp = srcp.reshape(_NW, _CPT, _CH)
    dstp = dstp.reshape(_NW, _CPT, _CH)
    ones16 = jnp.ones((_CH, 16), jnp.float32)
    zeros16 = jnp.zeros((_RPAD, 16), jnp.float32)
    zeros128 = jnp.zeros((_RPAD, _D), jnp.float32)

    deg_parts = _deg_sc(dstp, ones16, zeros16)
    h, g_pre = _pre_tc(x, fc0_w, fc0_b, bn0_g, bn0_b, conv_w)
    gs = _scale_tc(g_pre, deg_parts)
    parts = _gather_sc(gs, srcp, dstp, zeros128)
    return _post_tc(parts, deg_parts, gs, h, W_w, W_b, bn1_g, bn1_b,
                    fco_w, fco_b)
